# baseline (speedup 1.0000x reference)
import dataclasses
import functools

import jax
import jax.numpy as jnp
from jax import lax
from jax.experimental import pallas as pl
from jax.experimental.pallas import tpu as pltpu
from jax.experimental.pallas import tpu_sc as plsc

N = 10000
E = 320000
F = 128

NC = 2
NS = 16
NT = NC * NS
EPT = E // NT
RPT = 624
REXTRA = N - NS * RPT
BE = 80
NBLK = E // BE
NB0 = NBLK // NT
EALL = NB0 * BE

_ROW_CHUNKS = ((0, 128), (128, 128), (256, 128), (384, 128), (512, 112))

_MESH = plsc.VectorSubcoreMesh(core_axis_name="c", subcore_axis_name="s")

_SC_PARAMS = pltpu.CompilerParams()
if "needs_layout_passes" in pltpu.CompilerParams.__dataclass_fields__:
    _SC_PARAMS = dataclasses.replace(_SC_PARAMS, needs_layout_passes=False)


@functools.partial(
    pl.kernel,
    mesh=_MESH,
    compiler_params=_SC_PARAMS,
    out_type=jax.ShapeDtypeStruct((NT, 1, N), jnp.float32),
    scratch_types=[
        pltpu.VMEM((EPT,), jnp.int32),
        pltpu.VMEM((EPT,), jnp.float32),
        pltpu.VMEM((N,), jnp.float32),
    ],
)
def _sc_degree(dst_hbm, w_hbm, out_hbm, dst_v, w_v, deg_v):
    c = lax.axis_index("c")
    s = lax.axis_index("s")
    wid = s * NC + c
    base = wid * EPT

    zero16 = jnp.zeros((16,), jnp.float32)

    @pl.loop(0, N, step=16)
    def _(i):
        deg_v[pl.ds(i, 16)] = zero16

    pltpu.sync_copy(dst_hbm.at[pl.ds(base, EPT)], dst_v)
    pltpu.sync_copy(w_hbm.at[pl.ds(base, EPT)], w_v)

    @pl.loop(0, EPT, step=16)
    def _(e):
        idx = dst_v[pl.ds(e, 16)]
        w = w_v[pl.ds(e, 16)]
        plsc.addupdate_scatter(deg_v, [idx], w)

    pltpu.sync_copy(deg_v, out_hbm.at[wid, 0])


@functools.partial(
    pl.kernel,
    mesh=_MESH,
    compiler_params=_SC_PARAMS,
    out_type=jax.ShapeDtypeStruct((NC, N, F), jnp.float32),
    scratch_types=[
        pltpu.VMEM((3, 128), jnp.int32),
        pltpu.VMEM((3, 128), jnp.int32),
        pltpu.VMEM((3, 128), jnp.int32),
        pltpu.VMEM((3, 128), jnp.int32),
        pltpu.VMEM((BE, F), jnp.float32),
        pltpu.VMEM((BE, F), jnp.float32),
        pltpu.VMEM((BE, F), jnp.float32),
        pltpu.VMEM((BE, F), jnp.float32),
        pltpu.VMEM_SHARED((N, F), jnp.float32),
        pltpu.SemaphoreType.DMA,
        pltpu.SemaphoreType.DMA,
        pltpu.SemaphoreType.DMA,
        pltpu.SemaphoreType.DMA,
        pltpu.SemaphoreType.DMA,
        pltpu.SemaphoreType.DMA,
        pltpu.SemaphoreType.DMA,
        pltpu.SemaphoreType.DMA,
        pltpu.SemaphoreType.DMA,
        pltpu.SemaphoreType.DMA,
        pltpu.SemaphoreType.DMA,
        pltpu.SemaphoreType.DMA,
    ],
)
def _sc_propagate(g_hbm, epk_hbm, out_hbm,
                  ring0, ring1, ring2, ring3, rows0, rows1, rows2, rows3,
                  acc_sh,
                  sg0, sg1, sg2, sg3, ss0, ss1, ss2, ss3,
                  sp0, sp1, sp2, sp3):
    c = lax.axis_index("c")
    s = lax.axis_index("s")
    wid = s * NC + c
    blk_base = wid * NB0

    ring = (ring0, ring1, ring2, ring3)
    rows = (rows0, rows1, rows2, rows3)
    sg = (sg0, sg1, sg2, sg3)
    ss = (ss0, ss1, ss2, ss3)
    sp = (sp0, sp1, sp2, sp3)

    def pf(b, k):
        return pltpu.make_async_copy(epk_hbm.at[blk_base + b], ring[k], sp[k])

    zero16 = jnp.zeros((16,), jnp.float32)

    @pl.loop(0, BE)
    def _(r):
        for cc in range(0, F, 16):
            rows0[r, pl.ds(cc, 16)] = zero16

    rbase = s * RPT
    for off in range(0, RPT - BE + 1, BE):
        pltpu.sync_copy(rows0, acc_sh.at[pl.ds(rbase + off, BE)])
    _zrem = RPT % BE
    pltpu.sync_copy(rows0.at[pl.ds(0, _zrem)],
                    acc_sh.at[pl.ds(rbase + RPT - _zrem, _zrem)])

    @pl.when(s == NS - 1)
    def _():
        pltpu.sync_copy(rows0.at[pl.ds(0, REXTRA)],
                        acc_sh.at[pl.ds(NS * RPT, REXTRA)])

    for j in range(4):
        pf(j, j).start()

    plsc.subcore_barrier()

    def gather_d(b_k, k):
        return pltpu.make_async_copy(
            g_hbm.at[ring[k].at[0, pl.ds(0, BE)]], rows[k], sg[k])

    def scatter_start(k):
        for gph in range(BE // 16):
            idx = ring[k][1, pl.ds(gph * 16, 16)]
            pltpu.make_async_copy(rows[k].at[pl.ds(gph * 16, 16)],
                                  acc_sh.at[idx], ss[k]).start(add=True)

    def scatter_wait(k):
        pltpu.make_async_copy(g_hbm.at[pl.ds(0, BE)], rows[k], ss[k]).wait()

    def scale(k):
        @pl.loop(0, BE)
        def _(r):
            w_i = plsc.load_gather(ring[k], [jnp.full((16,), 2, jnp.int32),
                                             jnp.full((16,), r, jnp.int32)])
            w_b = plsc.bitcast(w_i, jnp.float32)
            for cc in range(0, F, 16):
                rows[k][r, pl.ds(cc, 16)] = rows[k][r, pl.ds(cc, 16)] * w_b

    def stage(sval, k, pf_start=True, do_gather=True, sc_wait=True):
        gather_d(sval, k).wait()
        if sc_wait:
            scatter_wait((k + 2) % 4)
        if do_gather:
            k2 = (k + 2) % 4
            pf(sval + 2, k2).wait()
            gather_d(sval + 2, k2).start()
        scale(k)
        scatter_start(k)
        if pf_start:
            pf(sval + 4, k).start()

    pf(0, 0).wait()
    gather_d(0, 0).start()
    pf(1, 1).wait()
    gather_d(1, 1).start()
    stage(0, 0, sc_wait=False)
    stage(1, 1, sc_wait=False)

    @pl.loop(2, 118, step=4)
    def _(b):
        stage(b + 0, 2)
        stage(b + 1, 3)
        stage(b + 2, 0)
        stage(b + 3, 1)

    stage(118, 2)
    stage(119, 3)
    stage(120, 0)
    stage(121, 1, pf_start=False)
    stage(122, 2, pf_start=False)
    stage(123, 3, pf_start=False, do_gather=False)
    stage(124, 0, pf_start=False, do_gather=False)
    scatter_wait(3)
    scatter_wait(0)

    plsc.subcore_barrier()

    for off, sz in _ROW_CHUNKS:
        pltpu.sync_copy(acc_sh.at[pl.ds(rbase + off, sz)],
                        out_hbm.at[c, pl.ds(rbase + off, sz)])

    @pl.when(s == NS - 1)
    def _():
        pltpu.sync_copy(acc_sh.at[pl.ds(NS * RPT, REXTRA)],
                        out_hbm.at[c, pl.ds(NS * RPT, REXTRA)])


def _mm1_body(x_ref, w_ref, o_ref):
    o_ref[...] = jnp.dot(x_ref[...], w_ref[...],
                         preferred_element_type=jnp.float32)


def _scale_body(h_ref, degt_ref, g_ref, dinv_ref):
    deg = jnp.sum(degt_ref[...], axis=1, keepdims=True) + 1.0
    safe = jnp.where(deg > 0, deg, 1.0)
    dinv = jnp.where(deg > 0, lax.rsqrt(safe), 0.0)
    dinv_ref[...] = dinv
    g_ref[...] = h_ref[...] * dinv


def _final_body(accp_ref, g_ref, dinv_ref, b1_ref, gamma_ref, beta_ref,
                wlin_ref, blin_ref, o_ref):
    acc = accp_ref[0] + accp_ref[1] + g_ref[...]
    agg = acc * dinv_ref[...] + b1_ref[...]
    a = jnp.maximum(agg, 0.0)
    mean = jnp.mean(a, axis=0, keepdims=True)
    var = jnp.mean(a * a, axis=0, keepdims=True) - mean * mean
    cscale = gamma_ref[...] * lax.rsqrt(var + 1e-5)
    a_bn = (a - mean) * cscale + beta_ref[...]
    o_ref[...] = jnp.dot(a_bn, wlin_ref[...],
                         preferred_element_type=jnp.float32) + blin_ref[...]


def kernel(x, edge_index, edge_weight, W1, b1, gamma, beta, Wlin, blin):
    src = edge_index[0]
    dst = edge_index[1]

    deg_parts = _sc_degree(dst, edge_weight).reshape(NT, N)
    h = pl.pallas_call(
        _mm1_body,
        out_shape=jax.ShapeDtypeStruct((N, F), jnp.float32),
    )(x, W1)

    g, dinv = pl.pallas_call(
        _scale_body,
        out_shape=[
            jax.ShapeDtypeStruct((N, F), jnp.float32),
            jax.ShapeDtypeStruct((N, 1), jnp.float32),
        ],
    )(h, deg_parts.T)

    wbits = lax.bitcast_convert_type(edge_weight, jnp.int32)
    padc = ((0, 0), (0, 128 - BE))
    epk = jnp.stack([jnp.pad(src.reshape(NBLK, BE), padc),
                     jnp.pad(dst.reshape(NBLK, BE), padc),
                     jnp.pad(wbits.reshape(NBLK, BE), padc)],
                    axis=1)
    acc_parts = _sc_propagate(g, epk)

    out = pl.pallas_call(
        _final_body,
        out_shape=jax.ShapeDtypeStruct((N, F), jnp.float32),
    )(acc_parts, g, dinv, b1.reshape(1, F), gamma.reshape(1, F),
      beta.reshape(1, F), Wlin, blin.reshape(1, F))
    return out


# speedup vs baseline: 1.0437x; 1.0437x over previous
import dataclasses
import functools

import jax
import jax.numpy as jnp
from jax import lax
from jax.experimental import pallas as pl
from jax.experimental.pallas import tpu as pltpu
from jax.experimental.pallas import tpu_sc as plsc

N = 10000
E = 320000
F = 128

NC = 2
NS = 16
NT = NC * NS
EPT = E // NT
RPT = 624
REXTRA = N - NS * RPT
BE = 80
NBLK = E // BE
NB0 = NBLK // NT
EALL = NB0 * BE

_ROW_CHUNKS = ((0, 128), (128, 128), (256, 128), (384, 128), (512, 112))

_MESH = plsc.VectorSubcoreMesh(core_axis_name="c", subcore_axis_name="s")

_SC_PARAMS = pltpu.CompilerParams()
if "needs_layout_passes" in pltpu.CompilerParams.__dataclass_fields__:
    _SC_PARAMS = dataclasses.replace(_SC_PARAMS, needs_layout_passes=False)


@functools.partial(
    pl.kernel,
    mesh=_MESH,
    compiler_params=_SC_PARAMS,
    out_type=jax.ShapeDtypeStruct((NT, 1, N), jnp.float32),
    scratch_types=[
        pltpu.VMEM((EPT,), jnp.int32),
        pltpu.VMEM((EPT,), jnp.float32),
        pltpu.VMEM((N,), jnp.float32),
        pltpu.SemaphoreType.DMA,
    ],
)
def _sc_degree(dst_hbm, w_hbm, out_hbm, dst_v, w_v, deg_v, sem):
    c = lax.axis_index("c")
    s = lax.axis_index("s")
    wid = s * NC + c
    base = wid * EPT

    st1 = pltpu.make_async_copy(dst_hbm.at[pl.ds(base, EPT)], dst_v, sem)
    st2 = pltpu.make_async_copy(w_hbm.at[pl.ds(base, EPT)], w_v, sem)
    st1.start()
    st2.start()

    zero16 = jnp.zeros((16,), jnp.float32)

    @pl.loop(0, N, step=16)
    def _(i):
        deg_v[pl.ds(i, 16)] = zero16

    st1.wait()
    st2.wait()

    @pl.loop(0, EPT, step=16)
    def _(e):
        idx = dst_v[pl.ds(e, 16)]
        w = w_v[pl.ds(e, 16)]
        plsc.addupdate_scatter(deg_v, [idx], w)

    pltpu.sync_copy(deg_v, out_hbm.at[wid, 0])


@functools.partial(
    pl.kernel,
    mesh=_MESH,
    compiler_params=_SC_PARAMS,
    out_type=jax.ShapeDtypeStruct((NC, N, F), jnp.float32),
    scratch_types=[
        pltpu.VMEM((3, 128), jnp.int32),
        pltpu.VMEM((3, 128), jnp.int32),
        pltpu.VMEM((3, 128), jnp.int32),
        pltpu.VMEM((3, 128), jnp.int32),
        pltpu.VMEM((BE, F), jnp.float32),
        pltpu.VMEM((BE, F), jnp.float32),
        pltpu.VMEM((BE, F), jnp.float32),
        pltpu.VMEM((BE, F), jnp.float32),
        pltpu.VMEM_SHARED((N, F), jnp.float32),
        pltpu.SemaphoreType.DMA,
        pltpu.SemaphoreType.DMA,
        pltpu.SemaphoreType.DMA,
        pltpu.SemaphoreType.DMA,
        pltpu.SemaphoreType.DMA,
        pltpu.SemaphoreType.DMA,
        pltpu.SemaphoreType.DMA,
        pltpu.SemaphoreType.DMA,
        pltpu.SemaphoreType.DMA,
        pltpu.SemaphoreType.DMA,
        pltpu.SemaphoreType.DMA,
        pltpu.SemaphoreType.DMA,
    ],
)
def _sc_propagate(g_hbm, epk_hbm, out_hbm,
                  ring0, ring1, ring2, ring3, rows0, rows1, rows2, rows3,
                  acc_sh,
                  sg0, sg1, sg2, sg3, ss0, ss1, ss2, ss3,
                  sp0, sp1, sp2, sp3):
    c = lax.axis_index("c")
    s = lax.axis_index("s")
    wid = s * NC + c
    blk_base = wid * NB0

    ring = (ring0, ring1, ring2, ring3)
    rows = (rows0, rows1, rows2, rows3)
    sg = (sg0, sg1, sg2, sg3)
    ss = (ss0, ss1, ss2, ss3)
    sp = (sp0, sp1, sp2, sp3)

    def pf(b, k):
        return pltpu.make_async_copy(epk_hbm.at[blk_base + b], ring[k], sp[k])

    zero16 = jnp.zeros((16,), jnp.float32)

    @pl.loop(0, BE)
    def _(r):
        for cc in range(0, F, 16):
            rows0[r, pl.ds(cc, 16)] = zero16

    rbase = s * RPT
    for off in range(0, RPT - BE + 1, BE):
        pltpu.sync_copy(rows0, acc_sh.at[pl.ds(rbase + off, BE)])
    _zrem = RPT % BE
    pltpu.sync_copy(rows0.at[pl.ds(0, _zrem)],
                    acc_sh.at[pl.ds(rbase + RPT - _zrem, _zrem)])

    @pl.when(s == NS - 1)
    def _():
        pltpu.sync_copy(rows0.at[pl.ds(0, REXTRA)],
                        acc_sh.at[pl.ds(NS * RPT, REXTRA)])

    for j in range(4):
        pf(j, j).start()

    plsc.subcore_barrier()

    def gather_d(b_k, k):
        return pltpu.make_async_copy(
            g_hbm.at[ring[k].at[0, pl.ds(0, BE)]], rows[k], sg[k])

    def scatter_start(k):
        for gph in range(BE // 16):
            idx = ring[k][1, pl.ds(gph * 16, 16)]
            pltpu.make_async_copy(rows[k].at[pl.ds(gph * 16, 16)],
                                  acc_sh.at[idx], ss[k]).start(add=True)

    def scatter_wait(k):
        pltpu.make_async_copy(g_hbm.at[pl.ds(0, BE)], rows[k], ss[k]).wait()

    def scale(k):
        @pl.loop(0, BE)
        def _(r):
            w_i = plsc.load_gather(ring[k], [jnp.full((16,), 2, jnp.int32),
                                             jnp.full((16,), r, jnp.int32)])
            w_b = plsc.bitcast(w_i, jnp.float32)
            for cc in range(0, F, 16):
                rows[k][r, pl.ds(cc, 16)] = rows[k][r, pl.ds(cc, 16)] * w_b

    def stage(sval, k, pf_start=True, do_gather=True, sc_wait=True):
        gather_d(sval, k).wait()
        if sc_wait:
            scatter_wait((k + 2) % 4)
        if do_gather:
            k2 = (k + 2) % 4
            pf(sval + 2, k2).wait()
            gather_d(sval + 2, k2).start()
        scale(k)
        scatter_start(k)
        if pf_start:
            pf(sval + 4, k).start()

    pf(0, 0).wait()
    gather_d(0, 0).start()
    pf(1, 1).wait()
    gather_d(1, 1).start()
    stage(0, 0, sc_wait=False)
    stage(1, 1, sc_wait=False)

    @pl.loop(2, 118, step=4)
    def _(b):
        stage(b + 0, 2)
        stage(b + 1, 3)
        stage(b + 2, 0)
        stage(b + 3, 1)

    stage(118, 2)
    stage(119, 3)
    stage(120, 0)
    stage(121, 1, pf_start=False)
    stage(122, 2, pf_start=False)
    stage(123, 3, pf_start=False, do_gather=False)
    stage(124, 0, pf_start=False, do_gather=False)
    scatter_wait(3)
    scatter_wait(0)

    plsc.subcore_barrier()

    for off, sz in _ROW_CHUNKS:
        pltpu.sync_copy(acc_sh.at[pl.ds(rbase + off, sz)],
                        out_hbm.at[c, pl.ds(rbase + off, sz)])

    @pl.when(s == NS - 1)
    def _():
        pltpu.sync_copy(acc_sh.at[pl.ds(NS * RPT, REXTRA)],
                        out_hbm.at[c, pl.ds(NS * RPT, REXTRA)])


def _scale_body(x_ref, w_ref, degt_ref, g_ref, dinv_ref):
    deg = jnp.sum(degt_ref[...], axis=1, keepdims=True) + 1.0
    safe = jnp.where(deg > 0, deg, 1.0)
    dinv = jnp.where(deg > 0, lax.rsqrt(safe), 0.0)
    dinv_ref[...] = dinv
    h = jnp.dot(x_ref[...], w_ref[...], preferred_element_type=jnp.float32)
    g_ref[...] = h * dinv


def _final_body(accp_ref, g_ref, dinv_ref, b1_ref, gamma_ref, beta_ref,
                wlin_ref, blin_ref, o_ref):
    acc = accp_ref[0] + accp_ref[1] + g_ref[...]
    agg = acc * dinv_ref[...] + b1_ref[...]
    a = jnp.maximum(agg, 0.0)
    mean = jnp.mean(a, axis=0, keepdims=True)
    var = jnp.mean(a * a, axis=0, keepdims=True) - mean * mean
    cscale = gamma_ref[...] * lax.rsqrt(var + 1e-5)
    a_bn = (a - mean) * cscale + beta_ref[...]
    o_ref[...] = jnp.dot(a_bn, wlin_ref[...],
                         preferred_element_type=jnp.float32) + blin_ref[...]


def kernel(x, edge_index, edge_weight, W1, b1, gamma, beta, Wlin, blin):
    src = edge_index[0]
    dst = edge_index[1]

    deg_parts = _sc_degree(dst, edge_weight).reshape(NT, N)

    g, dinv = pl.pallas_call(
        _scale_body,
        out_shape=[
            jax.ShapeDtypeStruct((N, F), jnp.float32),
            jax.ShapeDtypeStruct((N, 1), jnp.float32),
        ],
    )(x, W1, deg_parts.T)

    wbits = lax.bitcast_convert_type(edge_weight, jnp.int32)
    padc = ((0, 0), (0, 128 - BE))
    epk = jnp.stack([jnp.pad(src.reshape(NBLK, BE), padc),
                     jnp.pad(dst.reshape(NBLK, BE), padc),
                     jnp.pad(wbits.reshape(NBLK, BE), padc)],
                    axis=1)
    acc_parts = _sc_propagate(g, epk)

    out = pl.pallas_call(
        _final_body,
        out_shape=jax.ShapeDtypeStruct((N, F), jnp.float32),
    )(acc_parts, g, dinv, b1.reshape(1, F), gamma.reshape(1, F),
      beta.reshape(1, F), Wlin, blin.reshape(1, F))
    return out
